# Initial kernel scaffold; baseline (speedup 1.0000x reference)
import jax
import jax.numpy as jnp
from jax.experimental import pallas as pl


def kernel(x, edge_index, W1, b1, g1, be1, W2, b2, g2, be2, W3, b3, g3, be3, W4, b4, g4, be4):
    raise NotImplementedError("write your pallas kernel here")


# speedup vs baseline: 12.7706x; 12.7706x over previous
import functools

import jax
import jax.numpy as jnp
from jax import lax
from jax.experimental import pallas as pl
from jax.experimental.pallas import tpu as pltpu
from jax.experimental.pallas import tpu_sc as plsc

N = 10000
E = 320000
D = 128
H = 128
NC = 2
NS = 16
NW = NC * NS
CHUNK = 128
NCHUNKS = E // CHUNK
ITERS = (NCHUNKS + NW - 1) // NW
NPAD = 10240
RPT = NPAD // NS
EPS = 1e-5

_MESH = plsc.VectorSubcoreMesh(
    core_axis_name="c", subcore_axis_name="s", num_cores=NC, num_subcores=NS)


def _zero_fill(ref, nrows, ncols):
  z16 = jnp.zeros((16,), jnp.float32)

  def body(i, _):
    for j in range(ncols // 16):
      ref[i, pl.ds(j * 16, 16)] = z16
    return 0

  lax.fori_loop(0, nrows, body, 0)


@functools.partial(
    pl.kernel,
    out_type=jax.ShapeDtypeStruct((NC * NPAD, 16), jnp.float32),
    mesh=_MESH,
    scratch_types=[
        pltpu.VMEM_SHARED((NPAD, 16), jnp.float32),
        pltpu.VMEM((1, CHUNK), jnp.int32),
        pltpu.VMEM((CHUNK, 16), jnp.float32),
        pltpu.VMEM((RPT, 16), jnp.float32),
    ],
)
def _sc_degree(dst_hbm, out_hbm, hist, didx, ones, zeros):
  cid = lax.axis_index("c")
  sid = lax.axis_index("s")
  wid = sid * NC + cid

  one16 = jnp.ones((16,), jnp.float32)

  def fill_ones(i, _):
    ones[i, pl.ds(0, 16)] = one16
    return 0

  lax.fori_loop(0, CHUNK, fill_ones, 0)
  _zero_fill(zeros, RPT, 16)
  pltpu.sync_copy(zeros, hist.at[pl.ds(sid * RPT, RPT)])
  plsc.subcore_barrier()

  def body(it, _):
    g = it * NW + wid

    @pl.when(g < NCHUNKS)
    def _():
      pltpu.sync_copy(dst_hbm.at[pl.ds(g * CHUNK, CHUNK)], didx.at[0])
      pltpu.sync_copy(ones, hist.at[didx.at[0]], add=True)

    return 0

  lax.fori_loop(0, ITERS, body, 0)
  plsc.subcore_barrier()
  pltpu.sync_copy(hist.at[pl.ds(sid * RPT, RPT)],
                  out_hbm.at[pl.ds(cid * NPAD + sid * RPT, RPT)])


@functools.partial(
    pl.kernel,
    out_type=jax.ShapeDtypeStruct((NC * NPAD, H), jnp.float32),
    mesh=_MESH,
    scratch_types=[
        pltpu.VMEM_SHARED((NPAD, H), jnp.float32),
        pltpu.VMEM((2, CHUNK), jnp.int32),
        pltpu.VMEM((CHUNK, H), jnp.float32),
        pltpu.VMEM((RPT // 5, H), jnp.float32),
        pltpu.SemaphoreType.DMA,
    ],
)
def _sc_aggregate(u_hbm, src_hbm, dst_hbm, out_hbm, acc, idx, rows, zeros, sem):
  cid = lax.axis_index("c")
  sid = lax.axis_index("s")
  wid = sid * NC + cid

  zr = RPT // 5
  _zero_fill(zeros, zr, H)
  for k in range(5):
    pltpu.sync_copy(zeros, acc.at[pl.ds(sid * RPT + k * zr, zr)])
  plsc.subcore_barrier()

  def body(it, _):
    g = it * NW + wid

    @pl.when(g < NCHUNKS)
    def _():
      base = g * CHUNK
      pltpu.sync_copy(src_hbm.at[pl.ds(base, CHUNK)], idx.at[0])
      pltpu.sync_copy(dst_hbm.at[pl.ds(base, CHUNK)], idx.at[1])
      pltpu.async_copy(u_hbm.at[idx.at[0]], rows, sem).wait()
      pltpu.sync_copy(rows, acc.at[idx.at[1]], add=True)

    return 0

  lax.fori_loop(0, ITERS, body, 0)
  plsc.subcore_barrier()
  pltpu.sync_copy(acc.at[pl.ds(sid * RPT, RPT)],
                  out_hbm.at[pl.ds(cid * NPAD + sid * RPT, RPT)])


def _tc_first_body(hist_ref, x_ref, w_ref, dinv_ref, u_ref):
  deg = hist_ref[pl.ds(0, N), 0:1] + hist_ref[pl.ds(NPAD, N), 0:1] + 1.0
  dinv = lax.rsqrt(deg)
  dinv_ref[...] = dinv
  u_ref[...] = jnp.dot(x_ref[...], w_ref[...],
                       preferred_element_type=jnp.float32) * dinv


def _tc_mid_body(part_ref, u_ref, dinv_ref, b_ref, g_ref, be_ref, w_ref,
                 unext_ref):
  dinv = dinv_ref[...]
  t = (part_ref[pl.ds(0, N), :] + part_ref[pl.ds(NPAD, N), :] + u_ref[...]) * dinv
  y = jnp.maximum(t + b_ref[...], 0.0)
  mu = jnp.mean(y, axis=0, keepdims=True)
  var = jnp.mean((y - mu) ** 2, axis=0, keepdims=True)
  z = (y - mu) * lax.rsqrt(var + EPS) * g_ref[...] + be_ref[...]
  unext_ref[...] = jnp.dot(z, w_ref[...],
                           preferred_element_type=jnp.float32) * dinv


def _tc_last_body(part_ref, u_ref, dinv_ref, b_ref, g_ref, be_ref, out_ref):
  dinv = dinv_ref[...]
  t = (part_ref[pl.ds(0, N), :] + part_ref[pl.ds(NPAD, N), :] + u_ref[...]) * dinv
  y = jnp.maximum(t + b_ref[...], 0.0)
  mu = jnp.mean(y, axis=0, keepdims=True)
  var = jnp.mean((y - mu) ** 2, axis=0, keepdims=True)
  out_ref[...] = (y - mu) * lax.rsqrt(var + EPS) * g_ref[...] + be_ref[...]


_tc_first = pl.pallas_call(
    _tc_first_body,
    out_shape=(jax.ShapeDtypeStruct((N, 1), jnp.float32),
               jax.ShapeDtypeStruct((N, H), jnp.float32)),
)

_tc_mid = pl.pallas_call(
    _tc_mid_body,
    out_shape=jax.ShapeDtypeStruct((N, H), jnp.float32),
)

_tc_last = pl.pallas_call(
    _tc_last_body,
    out_shape=jax.ShapeDtypeStruct((N, H), jnp.float32),
)


def kernel(x, edge_index, W1, b1, g1, be1, W2, b2, g2, be2, W3, b3, g3, be3,
           W4, b4, g4, be4):
  src = edge_index[0]
  dst = edge_index[1]

  hist = _sc_degree(dst)
  dinv, u = _tc_first(hist, x, W1)

  params = [(b1, g1, be1), (b2, g2, be2), (b3, g3, be3), (b4, g4, be4)]
  nxt = [W2, W3, W4]
  for i in range(4):
    b, g, be = params[i]
    part = _sc_aggregate(u, src, dst)
    b2d = b.reshape(1, H)
    g2d = g.reshape(1, H)
    be2d = be.reshape(1, H)
    if i < 3:
      u = _tc_mid(part, u, dinv, b2d, g2d, be2d, nxt[i])
    else:
      u = _tc_last(part, u, dinv, b2d, g2d, be2d)
  return u
